# baseline (speedup 1.0000x reference)
import jax
import jax.numpy as jnp
from jax import lax
from jax.experimental import pallas as pl
from jax.experimental.pallas import tpu as pltpu
from jax.experimental.pallas import tpu_sc as plsc

NUM_EMB = 100000
DIM = 300
BATCH = 4096
SEQ = 50
TOTAL = BATCH * SEQ
NUM_WORKERS = 32
PER_WORKER = TOTAL // NUM_WORKERS
CHUNK = 128
NCHUNK = PER_WORKER // CHUNK


def _emb_body(idx_hbm, table_hbm, out_hbm, idx_v, rows_v, gsem):
    wid = lax.axis_index("s") * 2 + lax.axis_index("c")
    base = wid * PER_WORKER

    pltpu.sync_copy(idx_hbm.at[wid], idx_v)

    @pl.loop(0, NCHUNK)
    def _(i):
        pltpu.async_copy(table_hbm.at[idx_v.at[i]], rows_v, gsem).wait()
        pltpu.sync_copy(rows_v, out_hbm.at[pl.ds(base + i * CHUNK, CHUNK)])


@jax.jit
def kernel(x, W):
    idx = x.reshape(NUM_WORKERS, NCHUNK, CHUNK)
    mesh = plsc.VectorSubcoreMesh(core_axis_name="c", subcore_axis_name="s")
    out = pl.kernel(
        _emb_body,
        out_type=jax.ShapeDtypeStruct((TOTAL, DIM), jnp.float32),
        mesh=mesh,
        scratch_types=[
            pltpu.VMEM((NCHUNK, CHUNK), jnp.int32),
            pltpu.VMEM((CHUNK, DIM), jnp.float32),
            pltpu.SemaphoreType.DMA,
        ],
        compiler_params=pltpu.CompilerParams(use_tc_tiling_on_sc=False),
    )(idx, W)
    return out.reshape(BATCH, SEQ, DIM)


# speedup vs baseline: 1.0179x; 1.0179x over previous
import jax
import jax.numpy as jnp
from jax import lax
from jax.experimental import pallas as pl
from jax.experimental.pallas import tpu as pltpu
from jax.experimental.pallas import tpu_sc as plsc

NUM_EMB = 100000
DIM = 300
DIMP = 304
BATCH = 4096
SEQ = 50
TOTAL = BATCH * SEQ
NUM_WORKERS = 32
PER_WORKER = TOTAL // NUM_WORKERS
CHUNK = 128
NCHUNK = PER_WORKER // CHUNK


def _emb_body(idx_hbm, table_hbm, out_hbm, idx_v, rows0, rows1, gsem0, gsem1, osem0, osem1):
    wid = lax.axis_index("s") * 2 + lax.axis_index("c")
    base = wid * PER_WORKER

    pltpu.sync_copy(idx_hbm.at[wid], idx_v)

    rows = (rows0, rows1)
    gsems = (gsem0, gsem1)
    osems = (osem0, osem1)

    pltpu.async_copy(table_hbm.at[idx_v.at[0]], rows0, gsem0)

    @pl.loop(0, NCHUNK, step=2)
    def _(i0):
        for b in range(2):
            i = i0 + b
            pltpu.make_async_copy(table_hbm.at[idx_v.at[i]], rows[b], gsems[b]).wait()

            @pl.when(jnp.logical_and(i >= 1, i + 1 < NCHUNK))
            def _():
                pltpu.make_async_copy(
                    rows[1 - b],
                    out_hbm.at[pl.ds(base + (i - 1) * CHUNK, CHUNK)],
                    osems[1 - b],
                ).wait()

            @pl.when(i + 1 < NCHUNK)
            def _():
                pltpu.async_copy(table_hbm.at[idx_v.at[i + 1]], rows[1 - b], gsems[1 - b])

            pltpu.async_copy(rows[b], out_hbm.at[pl.ds(base + i * CHUNK, CHUNK)], osems[b])

    pltpu.make_async_copy(
        rows[0], out_hbm.at[pl.ds(base + (NCHUNK - 2) * CHUNK, CHUNK)], osems[0]
    ).wait()
    pltpu.make_async_copy(
        rows[1], out_hbm.at[pl.ds(base + (NCHUNK - 1) * CHUNK, CHUNK)], osems[1]
    ).wait()


@jax.jit
def kernel(x, W):
    idx = x.reshape(NUM_WORKERS, NCHUNK, CHUNK)
    table = jnp.pad(W, ((0, 0), (0, DIMP - DIM)))
    mesh = plsc.VectorSubcoreMesh(core_axis_name="c", subcore_axis_name="s")
    out = pl.kernel(
        _emb_body,
        out_type=jax.ShapeDtypeStruct((TOTAL, DIMP), jnp.float32),
        mesh=mesh,
        scratch_types=[
            pltpu.VMEM((NCHUNK, CHUNK), jnp.int32),
            pltpu.VMEM((CHUNK, DIMP), jnp.float32),
            pltpu.VMEM((CHUNK, DIMP), jnp.float32),
            pltpu.SemaphoreType.DMA,
            pltpu.SemaphoreType.DMA,
            pltpu.SemaphoreType.DMA,
            pltpu.SemaphoreType.DMA,
        ],
        compiler_params=pltpu.CompilerParams(use_tc_tiling_on_sc=False),
    )(idx, table)
    return out[:, :DIM].reshape(BATCH, SEQ, DIM)


# speedup vs baseline: 1.2512x; 1.2293x over previous
import jax
import jax.numpy as jnp
from jax import lax
from jax.experimental import pallas as pl
from jax.experimental.pallas import tpu as pltpu
from jax.experimental.pallas import tpu_sc as plsc

NUM_EMB = 100000
DIM = 300
DIMP = 304
BATCH = 4096
SEQ = 50
TOTAL = BATCH * SEQ
NUM_WORKERS = 32
PER_WORKER = TOTAL // NUM_WORKERS
CHUNK = 128
NCHUNK = PER_WORKER // CHUNK


def _emb_body(idx_hbm, table_hbm, out_hbm, idx_v, rows0, rows1, gsem0, gsem1, osem0, osem1):
    wid = lax.axis_index("s") * 2 + lax.axis_index("c")
    base = wid * PER_WORKER

    pltpu.sync_copy(idx_hbm.at[pl.ds(base, PER_WORKER)], idx_v)

    rows = (rows0, rows1)
    gsems = (gsem0, gsem1)
    osems = (osem0, osem1)

    pltpu.async_copy(table_hbm.at[idx_v.at[pl.ds(0, CHUNK)]], rows0, gsem0)

    @pl.loop(0, NCHUNK, step=2)
    def _(i0):
        for b in range(2):
            i = i0 + b
            pltpu.make_async_copy(
                table_hbm.at[idx_v.at[pl.ds(i * CHUNK, CHUNK)]], rows[b], gsems[b]
            ).wait()

            @pl.when(jnp.logical_and(i >= 1, i + 1 < NCHUNK))
            def _():
                pltpu.make_async_copy(
                    rows[1 - b],
                    out_hbm.at[pl.ds(base + (i - 1) * CHUNK, CHUNK)],
                    osems[1 - b],
                ).wait()

            @pl.when(i + 1 < NCHUNK)
            def _():
                pltpu.async_copy(
                    table_hbm.at[idx_v.at[pl.ds((i + 1) * CHUNK, CHUNK)]],
                    rows[1 - b],
                    gsems[1 - b],
                )

            pltpu.async_copy(
                rows[b], out_hbm.at[pl.ds(base + i * CHUNK, CHUNK)], osems[b]
            )

    pltpu.make_async_copy(
        rows[0], out_hbm.at[pl.ds(base + (NCHUNK - 2) * CHUNK, CHUNK)], osems[0]
    ).wait()
    pltpu.make_async_copy(
        rows[1], out_hbm.at[pl.ds(base + (NCHUNK - 1) * CHUNK, CHUNK)], osems[1]
    ).wait()


@jax.jit
def kernel(x, W):
    idx = x.reshape(TOTAL)
    table = jnp.pad(W, ((0, 0), (0, DIMP - DIM)))
    mesh = plsc.VectorSubcoreMesh(core_axis_name="c", subcore_axis_name="s")
    out = pl.kernel(
        _emb_body,
        out_type=jax.ShapeDtypeStruct((TOTAL, DIMP), jnp.float32),
        mesh=mesh,
        scratch_types=[
            pltpu.VMEM((PER_WORKER,), jnp.int32),
            pltpu.VMEM((CHUNK, DIMP), jnp.float32),
            pltpu.VMEM((CHUNK, DIMP), jnp.float32),
            pltpu.SemaphoreType.DMA,
            pltpu.SemaphoreType.DMA,
            pltpu.SemaphoreType.DMA,
            pltpu.SemaphoreType.DMA,
        ],
        compiler_params=pltpu.CompilerParams(use_tc_tiling_on_sc=False),
    )(idx, table)
    return out.reshape(BATCH, SEQ, DIMP)[:, :, :DIM]


# speedup vs baseline: 1.3802x; 1.1031x over previous
import jax
import jax.numpy as jnp
from jax import lax
from jax.experimental import pallas as pl
from jax.experimental.pallas import tpu as pltpu
from jax.experimental.pallas import tpu_sc as plsc

NUM_EMB = 100000
DIM = 300
DIMP = 384
BATCH = 4096
SEQ = 50
TOTAL = BATCH * SEQ
NUM_WORKERS = 32
PER_WORKER = TOTAL // NUM_WORKERS
CHUNK = 128
NCHUNK = PER_WORKER // CHUNK


def _emb_body(idx_hbm, table_hbm, out_hbm, idx_v, rows0, rows1, gsem0, gsem1, osem0, osem1):
    wid = lax.axis_index("s") * 2 + lax.axis_index("c")
    base = wid * PER_WORKER

    pltpu.sync_copy(idx_hbm.at[pl.ds(base, PER_WORKER)], idx_v)

    rows = (rows0, rows1)
    gsems = (gsem0, gsem1)
    osems = (osem0, osem1)

    pltpu.async_copy(table_hbm.at[idx_v.at[pl.ds(0, CHUNK)]], rows0, gsem0)

    @pl.loop(0, NCHUNK, step=2)
    def _(i0):
        for b in range(2):
            i = i0 + b
            pltpu.make_async_copy(
                table_hbm.at[idx_v.at[pl.ds(i * CHUNK, CHUNK)]], rows[b], gsems[b]
            ).wait()

            @pl.when(jnp.logical_and(i >= 1, i + 1 < NCHUNK))
            def _():
                pltpu.make_async_copy(
                    rows[1 - b],
                    out_hbm.at[pl.ds(base + (i - 1) * CHUNK, CHUNK)],
                    osems[1 - b],
                ).wait()

            @pl.when(i + 1 < NCHUNK)
            def _():
                pltpu.async_copy(
                    table_hbm.at[idx_v.at[pl.ds((i + 1) * CHUNK, CHUNK)]],
                    rows[1 - b],
                    gsems[1 - b],
                )

            pltpu.async_copy(
                rows[b], out_hbm.at[pl.ds(base + i * CHUNK, CHUNK)], osems[b]
            )

    pltpu.make_async_copy(
        rows[0], out_hbm.at[pl.ds(base + (NCHUNK - 2) * CHUNK, CHUNK)], osems[0]
    ).wait()
    pltpu.make_async_copy(
        rows[1], out_hbm.at[pl.ds(base + (NCHUNK - 1) * CHUNK, CHUNK)], osems[1]
    ).wait()


@jax.jit
def kernel(x, W):
    idx = x.reshape(TOTAL)
    table = jnp.pad(W, ((0, 0), (0, DIMP - DIM)))
    mesh = plsc.VectorSubcoreMesh(core_axis_name="c", subcore_axis_name="s")
    out = pl.kernel(
        _emb_body,
        out_type=jax.ShapeDtypeStruct((TOTAL, DIMP), jnp.float32),
        mesh=mesh,
        scratch_types=[
            pltpu.VMEM((PER_WORKER,), jnp.int32),
            pltpu.VMEM((CHUNK, DIMP), jnp.float32),
            pltpu.VMEM((CHUNK, DIMP), jnp.float32),
            pltpu.SemaphoreType.DMA,
            pltpu.SemaphoreType.DMA,
            pltpu.SemaphoreType.DMA,
            pltpu.SemaphoreType.DMA,
        ],
        compiler_params=pltpu.CompilerParams(use_tc_tiling_on_sc=True),
    )(idx, table)
    return out.reshape(BATCH, SEQ, DIMP)[:, :, :DIM]


# speedup vs baseline: 1.7161x; 1.2434x over previous
import jax
import jax.numpy as jnp
from jax import lax
from jax.experimental import pallas as pl
from jax.experimental.pallas import tpu as pltpu
from jax.experimental.pallas import tpu_sc as plsc

NUM_EMB = 100000
DIM = 300
DIMP = 384
BATCH = 4096
SEQ = 50
SEQP = 56
TOTAL = BATCH * SEQP
NUM_WORKERS = 32
PER_WORKER = TOTAL // NUM_WORKERS
CHUNK = 2 * SEQP
NCHUNK = PER_WORKER // CHUNK


def _emb_body(idx_hbm, table_hbm, out_hbm, idx_v, rows0, rows1, gsem0, gsem1, osem0, osem1):
    wid = lax.axis_index("s") * 2 + lax.axis_index("c")
    base = wid * PER_WORKER

    pltpu.sync_copy(idx_hbm.at[pl.ds(base, PER_WORKER)], idx_v)

    rows = (rows0, rows1)
    gsems = (gsem0, gsem1)
    osems = (osem0, osem1)

    pltpu.async_copy(table_hbm.at[idx_v.at[pl.ds(0, CHUNK)]], rows0, gsem0)

    @pl.loop(0, NCHUNK, step=2)
    def _(i0):
        for b in range(2):
            i = i0 + b
            pltpu.make_async_copy(
                table_hbm.at[idx_v.at[pl.ds(i * CHUNK, CHUNK)]], rows[b], gsems[b]
            ).wait()

            @pl.when(jnp.logical_and(i >= 1, i + 1 < NCHUNK))
            def _():
                pltpu.make_async_copy(
                    rows[1 - b],
                    out_hbm.at[pl.ds(base + (i - 1) * CHUNK, CHUNK)],
                    osems[1 - b],
                ).wait()

            @pl.when(i + 1 < NCHUNK)
            def _():
                pltpu.async_copy(
                    table_hbm.at[idx_v.at[pl.ds((i + 1) * CHUNK, CHUNK)]],
                    rows[1 - b],
                    gsems[1 - b],
                )

            pltpu.async_copy(
                rows[b], out_hbm.at[pl.ds(base + i * CHUNK, CHUNK)], osems[b]
            )

    pltpu.make_async_copy(
        rows[0], out_hbm.at[pl.ds(base + (NCHUNK - 2) * CHUNK, CHUNK)], osems[0]
    ).wait()
    pltpu.make_async_copy(
        rows[1], out_hbm.at[pl.ds(base + (NCHUNK - 1) * CHUNK, CHUNK)], osems[1]
    ).wait()


@jax.jit
def kernel(x, W):
    dummy = (
        jnp.arange(BATCH, dtype=jnp.int32)[:, None] * 6
        + jnp.arange(SEQP - SEQ, dtype=jnp.int32)[None, :]
    ) % NUM_EMB
    idx = jnp.concatenate([x, dummy], axis=1).reshape(TOTAL)
    table = jnp.pad(W, ((0, 0), (0, DIMP - DIM)))
    mesh = plsc.VectorSubcoreMesh(core_axis_name="c", subcore_axis_name="s")
    out = pl.kernel(
        _emb_body,
        out_type=jax.ShapeDtypeStruct((TOTAL, DIMP), jnp.float32),
        mesh=mesh,
        scratch_types=[
            pltpu.VMEM((PER_WORKER,), jnp.int32),
            pltpu.VMEM((CHUNK, DIMP), jnp.float32),
            pltpu.VMEM((CHUNK, DIMP), jnp.float32),
            pltpu.SemaphoreType.DMA,
            pltpu.SemaphoreType.DMA,
            pltpu.SemaphoreType.DMA,
            pltpu.SemaphoreType.DMA,
        ],
        compiler_params=pltpu.CompilerParams(use_tc_tiling_on_sc=True),
    )(idx, table)
    return out.reshape(BATCH, SEQP, DIMP)[:, :SEQ, :DIM]
